# baseline (speedup 1.0000x reference)
import jax
import jax.numpy as jnp
from jax import lax
from jax.experimental import pallas as pl
from jax.experimental.pallas import tpu as pltpu
from jax.experimental.pallas import tpu_sc as plsc

N = 16384
C = 16
DIM = 64
SPAN = 256
NW = 32
SPW = N // NW
ROWS_PER_CHUNK = 128
SPANS_PER_CHUNK = ROWS_PER_CHUNK // C
NCHUNK = SPW // SPANS_PER_CHUNK
L = 16


def _sc_pool_body(cand_hbm, scores_hbm, len_hbm, embed_hbm, out_hbm,
                  idx_v, scores_v, len_v, rows_v, tail_v, sem):
    wid = lax.axis_index("s") * 2 + lax.axis_index("c")

    pltpu.sync_copy(cand_hbm.at[wid], idx_v)
    pltpu.sync_copy(scores_hbm.at[wid], scores_v)
    pltpu.sync_copy(len_hbm.at[wid], len_v)

    def chunk(j, _):
        pltpu.async_copy(embed_hbm.at[idx_v.at[j]], rows_v, sem).wait()
        for jj in range(SPANS_PER_CHUNK):
            s = j * SPANS_PER_CHUNK + jj
            sidx = jnp.full((L,), s, dtype=jnp.int32)
            lb = plsc.load_gather(
                len_v, [sidx, jnp.zeros((L,), dtype=jnp.int32)])
            gate = jnp.where(lb > 0, jnp.float32(1.0), jnp.float32(0.0))
            accs = []
            for d in range(DIM // L):
                acc = jnp.zeros((L,), dtype=jnp.float32)
                for c in range(C):
                    bc = plsc.load_gather(
                        scores_v, [sidx, jnp.full((L,), c, dtype=jnp.int32)])
                    acc = acc + bc * rows_v[jj * C + c, pl.ds(d * L, L)]
                accs.append(acc)
            for d in range(DIM // L):
                tail_v[s, pl.ds(d * L, L)] = accs[d] * gate
        return ()

    lax.fori_loop(0, NCHUNK, chunk, (), unroll=False)

    pltpu.sync_copy(tail_v, out_hbm.at[wid])


@jax.jit
def _sc_pool(cand, scores_w, len_w, embed):
    mesh = plsc.VectorSubcoreMesh(core_axis_name="c", subcore_axis_name="s")
    f = pl.kernel(
        _sc_pool_body,
        out_type=jax.ShapeDtypeStruct((NW, SPW, DIM), jnp.float32),
        mesh=mesh,
        compiler_params=pltpu.CompilerParams(
            needs_layout_passes=False, use_tc_tiling_on_sc=False),
        scratch_types=[
            pltpu.VMEM((NCHUNK, ROWS_PER_CHUNK), jnp.int32),
            pltpu.VMEM((SPW, C), jnp.float32),
            pltpu.VMEM((SPW, 1), jnp.int32),
            pltpu.VMEM((ROWS_PER_CHUNK, DIM), jnp.float32),
            pltpu.VMEM((SPW, DIM), jnp.float32),
            pltpu.SemaphoreType.DMA,
        ],
    )
    return f(cand, scores_w, len_w, embed)


def _concat_body(span_ref, tail_ref, out_ref):
    out_ref[:, :SPAN] = span_ref[...]
    out_ref[:, SPAN:] = tail_ref[...]


@jax.jit
def _concat(span_flat, tail_flat):
    blk = 512
    return pl.pallas_call(
        _concat_body,
        grid=(N // blk,),
        in_specs=[
            pl.BlockSpec((blk, SPAN), lambda i: (i, 0)),
            pl.BlockSpec((blk, DIM), lambda i: (i, 0)),
        ],
        out_specs=pl.BlockSpec((blk, SPAN + DIM), lambda i: (i, 0)),
        out_shape=jax.ShapeDtypeStruct((N, SPAN + DIM), jnp.float32),
    )(span_flat, tail_flat)


def kernel(span_vecs, scores, mask_candidates, embed, candidates, len_candidates):
    cand = candidates.reshape(NW, NCHUNK, ROWS_PER_CHUNK)
    scores_w = scores.reshape(NW, SPW, C)
    len_w = len_candidates.reshape(NW, SPW, 1)
    tail = _sc_pool(cand, scores_w, len_w, embed)
    out = _concat(span_vecs.reshape(N, SPAN), tail.reshape(N, DIM))
    return out.reshape(1, N, SPAN + DIM)


# speedup vs baseline: 1.0305x; 1.0305x over previous
import jax
import jax.numpy as jnp
from jax import lax
from jax.experimental import pallas as pl
from jax.experimental.pallas import tpu as pltpu
from jax.experimental.pallas import tpu_sc as plsc

N = 16384
C = 16
DIM = 64
SPAN = 256
NW = 32
SPW = N // NW
RPC = 128
SPC = RPC // C
NCHUNK = SPW // SPC
L = 16


def _sc_pool_body(candT_hbm, scoresT_hbm, table_hbm, tailT_hbm,
                  candT_v, scoresT_v, pair_v, off_v, rows0_v, rows1_v,
                  tailT_v, sem0, sem1):
    wid = lax.axis_index("s") * 2 + lax.axis_index("c")
    base = wid * SPW

    pltpu.sync_copy(candT_hbm.at[:, pl.ds(base, SPW)], candT_v)
    pltpu.sync_copy(scoresT_hbm.at[:, pl.ds(base, SPW)], scoresT_v)

    iota = lax.iota(jnp.int32, L)

    def build(j, _):
        for jj in range(SPC):
            s = j * SPC + jj
            cv = plsc.load_gather(candT_v, [iota, jnp.full((L,), s, jnp.int32)])
            pair_v[j, pl.ds(jj * C, L)] = lax.shift_right_logical(cv, 1)
            off_v[j, pl.ds(jj * C, L)] = (cv & 1) * DIM
        return ()

    lax.fori_loop(0, NCHUNK, build, (), unroll=False)

    def gather_start(j, rows, s):
        pltpu.async_copy(table_hbm.at[pair_v.at[j]], rows, s)

    def gather_wait(j, rows, s):
        pltpu.make_async_copy(table_hbm.at[pair_v.at[j]], rows, s).wait()

    def compute(j, rows):
        for jj in range(SPC):
            s = j * SPC + jj
            sidx = jnp.full((L,), s, jnp.int32)
            accs = [jnp.zeros((L,), jnp.float32) for _ in range(DIM // L)]
            offv = off_v[j, pl.ds(jj * C, C)]
            for c in range(C):
                r = jj * C + c
                bc = plsc.load_gather(
                    scoresT_v, [jnp.full((L,), c, jnp.int32), sidx])
                off = offv[c]
                for d in range(DIM // L):
                    accs[d] = accs[d] + bc * rows[r, pl.ds(off + d * L, L)]
            for d in range(DIM // L):
                plsc.store_scatter(
                    tailT_v, [d * L + iota, sidx], accs[d])

    gather_start(0, rows0_v, sem0)

    def step(j2, _):
        j = j2 * 2
        gather_wait(j, rows0_v, sem0)
        gather_start(j + 1, rows1_v, sem1)
        compute(j, rows0_v)
        gather_wait(j + 1, rows1_v, sem1)

        @pl.when(j2 + 1 < NCHUNK // 2)
        def _():
            gather_start(j + 2, rows0_v, sem0)

        compute(j + 1, rows1_v)
        return ()

    lax.fori_loop(0, NCHUNK // 2, step, (), unroll=False)

    pltpu.sync_copy(tailT_v, tailT_hbm.at[:, pl.ds(base, SPW)])


@jax.jit
def _sc_pool(candT, scoresT, table):
    mesh = plsc.VectorSubcoreMesh(core_axis_name="c", subcore_axis_name="s")
    f = pl.kernel(
        _sc_pool_body,
        out_type=jax.ShapeDtypeStruct((DIM, N), jnp.float32),
        mesh=mesh,
        compiler_params=pltpu.CompilerParams(needs_layout_passes=False),
        scratch_types=[
            pltpu.VMEM((C, SPW), jnp.int32),
            pltpu.VMEM((C, SPW), jnp.float32),
            pltpu.VMEM((NCHUNK, RPC), jnp.int32),
            pltpu.VMEM((NCHUNK, RPC), jnp.int32),
            pltpu.VMEM((RPC, 2 * DIM), jnp.float32),
            pltpu.VMEM((RPC, 2 * DIM), jnp.float32),
            pltpu.VMEM((DIM, SPW), jnp.float32),
            pltpu.SemaphoreType.DMA,
            pltpu.SemaphoreType.DMA,
        ],
    )
    return f(candT, scoresT, table)


def _assemble_body(span_ref, tail_ref, len_ref, out_ref):
    sv = span_ref[...]
    out_ref[:SPAN, :] = sv.T
    gate = (len_ref[...] > 0).astype(jnp.float32)
    out_ref[SPAN:, :] = tail_ref[...] * gate


@jax.jit
def _assemble(span_flat, tailT, len_row):
    blk = 512
    return pl.pallas_call(
        _assemble_body,
        grid=(N // blk,),
        in_specs=[
            pl.BlockSpec((blk, SPAN), lambda i: (i, 0)),
            pl.BlockSpec((DIM, blk), lambda i: (0, i)),
            pl.BlockSpec((1, blk), lambda i: (0, i)),
        ],
        out_specs=pl.BlockSpec((SPAN + DIM, blk), lambda i: (0, i)),
        out_shape=jax.ShapeDtypeStruct((SPAN + DIM, N), jnp.float32),
    )(span_flat, tailT, len_row)


def kernel(span_vecs, scores, mask_candidates, embed, candidates, len_candidates):
    candT = candidates[0].T
    scoresT = scores[0].T
    table = embed.reshape(1000000 // 2, 2 * DIM)
    tailT = _sc_pool(candT, scoresT, table)
    outT = _assemble(span_vecs.reshape(N, SPAN), tailT,
                     len_candidates.reshape(1, N))
    return outT.T.reshape(1, N, SPAN + DIM)


# speedup vs baseline: 1.3177x; 1.2787x over previous
import jax
import jax.numpy as jnp
from jax import lax
from jax.experimental import pallas as pl
from jax.experimental.pallas import tpu as pltpu
from jax.experimental.pallas import tpu_sc as plsc

N = 16384
C = 16
DIM = 64
SPAN = 256
NW = 32
SPW = N // NW
RPC = 128
SPC = RPC // C
NCHUNK = SPW // SPC
L = 16


def _sc_pool_body(candT_hbm, scoresT_hbm, table_hbm, tailT_hbm,
                  candT_v, scoresT_v, pair_v, off_v, rows0_v, rows1_v,
                  tailT_v, sem0, sem1):
    wid = lax.axis_index("s") * 2 + lax.axis_index("c")
    base = wid * SPW

    pltpu.sync_copy(candT_hbm.at[:, pl.ds(base, SPW)], candT_v)
    pltpu.sync_copy(scoresT_hbm.at[:, pl.ds(base, SPW)], scoresT_v)

    iota = lax.iota(jnp.int32, L)

    def build(j, _):
        for jj in range(SPC):
            s = j * SPC + jj
            cv = plsc.load_gather(candT_v, [iota, jnp.full((L,), s, jnp.int32)])
            pair_v[j, pl.ds(jj * C, L)] = lax.shift_right_logical(cv, 1)
            off_v[j, pl.ds(jj * C, L)] = (cv & 1) * DIM
        return ()

    lax.fori_loop(0, NCHUNK, build, (), unroll=False)

    def gather_start(j, rows, s):
        pltpu.async_copy(table_hbm.at[pair_v.at[j]], rows, s)

    def gather_wait(j, rows, s):
        pltpu.make_async_copy(table_hbm.at[pair_v.at[j]], rows, s).wait()

    def compute(j, rows):
        for jj in range(SPC):
            s = j * SPC + jj
            sidx = jnp.full((L,), s, jnp.int32)
            accs = [jnp.zeros((L,), jnp.float32) for _ in range(DIM // L)]
            offv = off_v[j, pl.ds(jj * C, C)]
            for c in range(C):
                r = jj * C + c
                bc = plsc.load_gather(
                    scoresT_v, [jnp.full((L,), c, jnp.int32), sidx])
                off = offv[c]
                for d in range(DIM // L):
                    accs[d] = accs[d] + bc * rows[r, pl.ds(off + d * L, L)]
            for d in range(DIM // L):
                plsc.store_scatter(
                    tailT_v, [d * L + iota, sidx], accs[d])

    gather_start(0, rows0_v, sem0)

    def step(j2, _):
        j = j2 * 2
        gather_wait(j, rows0_v, sem0)
        gather_start(j + 1, rows1_v, sem1)
        compute(j, rows0_v)
        gather_wait(j + 1, rows1_v, sem1)

        @pl.when(j2 + 1 < NCHUNK // 2)
        def _():
            gather_start(j + 2, rows0_v, sem0)

        compute(j + 1, rows1_v)
        return ()

    lax.fori_loop(0, NCHUNK // 2, step, (), unroll=False)

    pltpu.sync_copy(tailT_v, tailT_hbm.at[:, pl.ds(base, SPW)])


@jax.jit
def _sc_pool(candT, scoresT, table):
    mesh = plsc.VectorSubcoreMesh(core_axis_name="c", subcore_axis_name="s")
    f = pl.kernel(
        _sc_pool_body,
        out_type=jax.ShapeDtypeStruct((DIM, N), jnp.float32),
        mesh=mesh,
        compiler_params=pltpu.CompilerParams(needs_layout_passes=False),
        scratch_types=[
            pltpu.VMEM((C, SPW), jnp.int32),
            pltpu.VMEM((C, SPW), jnp.float32),
            pltpu.VMEM((NCHUNK, RPC), jnp.int32),
            pltpu.VMEM((NCHUNK, RPC), jnp.int32),
            pltpu.VMEM((RPC, 2 * DIM), jnp.float32),
            pltpu.VMEM((RPC, 2 * DIM), jnp.float32),
            pltpu.VMEM((DIM, SPW), jnp.float32),
            pltpu.SemaphoreType.DMA,
            pltpu.SemaphoreType.DMA,
        ],
    )
    return f(candT, scoresT, table)


def _relayout_body(embT_ref, p_ref, out_ref):
    p = p_ref[...]
    nstrip = embT_ref.shape[1] // 128
    for k in range(nstrip):
        s = embT_ref[:, pl.ds(k * 128, 128)]
        mt = jax.lax.dot_general(p, s, (((0,), (1,)), ((), ())),
                                 preferred_element_type=jnp.float32)
        out_ref[pl.ds(k * DIM, DIM), :DIM] = mt[:DIM, :]
        out_ref[pl.ds(k * DIM, DIM), DIM:] = mt[DIM:, :]


@jax.jit
def _tc_relayout(embT, p):
    b = 1024
    nblk = -(-embT.shape[1] // (2 * b))
    return pl.pallas_call(
        _relayout_body,
        grid=(nblk,),
        in_specs=[
            pl.BlockSpec((DIM, 2 * b), lambda i: (0, i)),
            pl.BlockSpec((128, 128), lambda i: (0, 0)),
        ],
        out_specs=pl.BlockSpec((b, 2 * DIM), lambda i: (i, 0)),
        out_shape=jax.ShapeDtypeStruct((embT.shape[1] // 2, 2 * DIM),
                                       jnp.float32),
    )(embT, p)


def _assemble_body(span_ref, tail_ref, len_ref, out_ref):
    sv = span_ref[...]
    out_ref[:SPAN, :] = sv.T
    gate = (len_ref[...] > 0).astype(jnp.float32)
    out_ref[SPAN:, :] = tail_ref[...] * gate


@jax.jit
def _assemble(span_flat, tailT, len_row):
    blk = 512
    return pl.pallas_call(
        _assemble_body,
        grid=(N // blk,),
        in_specs=[
            pl.BlockSpec((blk, SPAN), lambda i: (i, 0)),
            pl.BlockSpec((DIM, blk), lambda i: (0, i)),
            pl.BlockSpec((1, blk), lambda i: (0, i)),
        ],
        out_specs=pl.BlockSpec((SPAN + DIM, blk), lambda i: (0, i)),
        out_shape=jax.ShapeDtypeStruct((SPAN + DIM, N), jnp.float32),
    )(span_flat, tailT, len_row)


def kernel(span_vecs, scores, mask_candidates, embed, candidates, len_candidates):
    candT = candidates[0].T
    scoresT = scores[0].T
    j = jnp.arange(128)
    perm = jnp.where(j % 2 == 0, j // 2, DIM + j // 2)
    p = (jnp.arange(128)[None, :] == perm[:, None]).astype(jnp.float32)
    table = _tc_relayout(embed.T, p)
    tailT = _sc_pool(candT, scoresT, table)
    outT = _assemble(span_vecs.reshape(N, SPAN), tailT,
                     len_candidates.reshape(1, N))
    return outT.T.reshape(1, N, SPAN + DIM)


# speedup vs baseline: 1.3200x; 1.0017x over previous
import jax
import jax.numpy as jnp
from jax import lax
from jax.experimental import pallas as pl
from jax.experimental.pallas import tpu as pltpu
from jax.experimental.pallas import tpu_sc as plsc

N = 16384
C = 16
DIM = 64
SPAN = 256
NW = 32
SPW = N // NW
RPC = 128
SPC = RPC // C
NCHUNK = SPW // SPC
L = 16


def _sc_pool_body(candT_hbm, scoresT_hbm, table_hbm, tailT_hbm,
                  candT_v, scoresT_v, pair_v, off_v, rows0_v, rows1_v,
                  tailT_v, sem0, sem1):
    wid = lax.axis_index("s") * 2 + lax.axis_index("c")
    base = wid * SPW

    pltpu.sync_copy(candT_hbm.at[:, pl.ds(base, SPW)], candT_v)
    pltpu.sync_copy(scoresT_hbm.at[:, pl.ds(base, SPW)], scoresT_v)

    iota = lax.iota(jnp.int32, L)

    def build(j, _):
        for jj in range(SPC):
            s = j * SPC + jj
            cv = plsc.load_gather(candT_v, [iota, jnp.full((L,), s, jnp.int32)])
            pair_v[j, pl.ds(jj * C, L)] = lax.shift_right_logical(cv, 1)
            off_v[j, pl.ds(jj * C, L)] = (cv & 1) * DIM
        return ()

    lax.fori_loop(0, NCHUNK, build, (), unroll=False)

    def gather_start(j, rows, s):
        pltpu.async_copy(table_hbm.at[pair_v.at[j]], rows, s)

    def gather_wait(j, rows, s):
        pltpu.make_async_copy(table_hbm.at[pair_v.at[j]], rows, s).wait()

    def compute(j, rows):
        for jj in range(SPC):
            s = j * SPC + jj
            sidx = jnp.full((L,), s, jnp.int32)
            accs = [jnp.zeros((L,), jnp.float32) for _ in range(DIM // L)]
            offv = off_v[j, pl.ds(jj * C, C)]
            for c in range(C):
                r = jj * C + c
                bc = plsc.load_gather(
                    scoresT_v, [jnp.full((L,), c, jnp.int32), sidx])
                off = offv[c]
                for d in range(DIM // L):
                    accs[d] = accs[d] + bc * rows[r, pl.ds(off + d * L, L)]
            for d in range(DIM // L):
                plsc.store_scatter(
                    tailT_v, [d * L + iota, sidx], accs[d])

    gather_start(0, rows0_v, sem0)

    def step(j2, _):
        j = j2 * 2
        gather_wait(j, rows0_v, sem0)
        gather_start(j + 1, rows1_v, sem1)
        compute(j, rows0_v)
        gather_wait(j + 1, rows1_v, sem1)

        @pl.when(j2 + 1 < NCHUNK // 2)
        def _():
            gather_start(j + 2, rows0_v, sem0)

        compute(j + 1, rows1_v)
        return ()

    lax.fori_loop(0, NCHUNK // 2, step, (), unroll=False)

    pltpu.sync_copy(tailT_v, tailT_hbm.at[:, pl.ds(base, SPW)])


@jax.jit
def _sc_pool(candT, scoresT, table):
    mesh = plsc.VectorSubcoreMesh(core_axis_name="c", subcore_axis_name="s")
    f = pl.kernel(
        _sc_pool_body,
        out_type=jax.ShapeDtypeStruct((DIM, N), jnp.float32),
        mesh=mesh,
        compiler_params=pltpu.CompilerParams(needs_layout_passes=False),
        scratch_types=[
            pltpu.VMEM((C, SPW), jnp.int32),
            pltpu.VMEM((C, SPW), jnp.float32),
            pltpu.VMEM((NCHUNK, RPC), jnp.int32),
            pltpu.VMEM((NCHUNK, RPC), jnp.int32),
            pltpu.VMEM((RPC, 2 * DIM), jnp.float32),
            pltpu.VMEM((RPC, 2 * DIM), jnp.float32),
            pltpu.VMEM((DIM, SPW), jnp.float32),
            pltpu.SemaphoreType.DMA,
            pltpu.SemaphoreType.DMA,
        ],
    )
    return f(candT, scoresT, table)


def _relayout_body(embT_ref, p_ref, out_ref):
    p = p_ref[...]
    nstrip = embT_ref.shape[1] // 128
    for k4 in range(nstrip // 4):
        s4 = jnp.concatenate(
            [embT_ref[:, pl.ds((k4 * 4 + a) * 128, 128)] for a in range(4)],
            axis=0)
        mt = jax.lax.dot_general(p, s4, (((0,), (1,)), ((), ())),
                                 preferred_element_type=jnp.float32)
        for a in range(4):
            k = k4 * 4 + a
            out_ref[pl.ds(k * DIM, DIM), :DIM] = mt[:DIM, a * DIM:(a + 1) * DIM]
            out_ref[pl.ds(k * DIM, DIM), DIM:] = mt[DIM:, a * DIM:(a + 1) * DIM]


@jax.jit
def _tc_relayout(embT, p):
    b = 1024
    nblk = -(-embT.shape[1] // (2 * b))
    return pl.pallas_call(
        _relayout_body,
        grid=(nblk,),
        in_specs=[
            pl.BlockSpec((DIM, 2 * b), lambda i: (0, i)),
            pl.BlockSpec((128, 128), lambda i: (0, 0)),
        ],
        out_specs=pl.BlockSpec((b, 2 * DIM), lambda i: (i, 0)),
        out_shape=jax.ShapeDtypeStruct((embT.shape[1] // 2, 2 * DIM),
                                       jnp.float32),
    )(embT, p)


def _assemble_body(span_ref, tail_ref, len_ref, out_ref):
    sv = span_ref[...]
    out_ref[:SPAN, :] = sv.T
    gate = (len_ref[...] > 0).astype(jnp.float32)
    out_ref[SPAN:, :] = tail_ref[...] * gate


@jax.jit
def _assemble(span_flat, tailT, len_row):
    blk = 512
    return pl.pallas_call(
        _assemble_body,
        grid=(N // blk,),
        in_specs=[
            pl.BlockSpec((blk, SPAN), lambda i: (i, 0)),
            pl.BlockSpec((DIM, blk), lambda i: (0, i)),
            pl.BlockSpec((1, blk), lambda i: (0, i)),
        ],
        out_specs=pl.BlockSpec((SPAN + DIM, blk), lambda i: (0, i)),
        out_shape=jax.ShapeDtypeStruct((SPAN + DIM, N), jnp.float32),
    )(span_flat, tailT, len_row)


def kernel(span_vecs, scores, mask_candidates, embed, candidates, len_candidates):
    candT = candidates[0].T
    scoresT = scores[0].T
    j = jnp.arange(128)
    perm = jnp.where(j % 2 == 0, j // 2, DIM + j // 2)
    p = (jnp.arange(128)[None, :] == perm[:, None]).astype(jnp.float32)
    table = _tc_relayout(embed.T, p)
    tailT = _sc_pool(candT, scoresT, table)
    outT = _assemble(span_vecs.reshape(N, SPAN), tailT,
                     len_candidates.reshape(1, N))
    return outT.T.reshape(1, N, SPAN + DIM)
